# baseline (speedup 1.0000x reference)
import functools

import jax
import jax.numpy as jnp
from jax import lax
from jax.experimental import pallas as pl
from jax.experimental.pallas import tpu as pltpu
from jax.experimental.pallas import tpu_sc as plsc

N_NODES = 10000
N_EDGES = 160000
D_FEAT = 256
D_OUT = 256
H = 128

NC = 2
NS = 16
NBK = NC * NS
CHUNK = 128
E_CAP = 164096
NB = 320
NP = 10240
NPC = NP // NC
TRASHL = NPC
NPCA = NPC + NS * 8


def _sc_body(x2_hbm, src2_hbm, dst_hbm, nch_hbm, base_hbm, zeros_hbm,
             ones_hbm, out_hbm,
             nchv, basev, srcv, dstv, rows, acc, sem):
    c = lax.axis_index("c")
    s = lax.axis_index("s")
    bucket = c * NS + s
    pltpu.sync_copy(nch_hbm, nchv)
    pltpu.sync_copy(base_hbm, basev)
    nch = nchv[pl.ds(bucket * 8, 16)][0]
    ebase = pl.multiple_of(basev[pl.ds(bucket * 8, 16)][0], CHUNK)

    for q in range(3):
        pltpu.sync_copy(zeros_hbm, rows)
        for z in range(NB // CHUNK):
            pltpu.sync_copy(rows, acc.at[pl.ds(s * NB + z * CHUNK, CHUNK)])
        pltpu.sync_copy(rows.at[pl.ds(0, NB % CHUNK)],
                        acc.at[pl.ds(s * NB + NB - NB % CHUNK, NB % CHUNK)])

        if q == 2:
            pltpu.sync_copy(ones_hbm, rows)

        def body(j, carry):
            off = ebase + j * CHUNK
            pltpu.sync_copy(dst_hbm.at[pl.ds(off, CHUNK)], dstv)
            if q < 2:
                pltpu.sync_copy(src2_hbm.at[pl.ds(q * E_CAP + off, CHUNK)],
                                srcv)
                pltpu.async_copy(x2_hbm.at[srcv], rows, sem).wait()
            pltpu.sync_copy(rows, acc.at[dstv], add=True)
            return carry

        lax.fori_loop(0, nch, body, 0)

        pltpu.sync_copy(acc.at[pl.ds(s * NB, NB)],
                        out_hbm.at[q, pl.ds(c * NPC + s * NB, NB)])


@functools.cache
def _build_sc_scatter():
    mesh = plsc.VectorSubcoreMesh(core_axis_name="c", subcore_axis_name="s",
                                  num_cores=NC, num_subcores=NS)
    return pl.kernel(
        _sc_body,
        out_type=jax.ShapeDtypeStruct((3, NP, H), jnp.float32),
        mesh=mesh,
        scratch_types=(
            pltpu.VMEM((8 * NBK + 16,), jnp.int32),
            pltpu.VMEM((8 * NBK + 16,), jnp.int32),
            pltpu.VMEM((CHUNK,), jnp.int32),
            pltpu.VMEM((CHUNK,), jnp.int32),
            pltpu.VMEM((CHUNK, H), jnp.float32),
            pltpu.VMEM_SHARED((NPCA, H), jnp.float32),
            pltpu.SemaphoreType.DMA,
        ),
    )


def _tc_body(x_ref, ns_ref, w_ref, b_ref, o_ref):
    r = 1.0 / jnp.maximum(ns_ref[2, :, 0:1], 1.0)
    h = jnp.concatenate([ns_ref[0], ns_ref[1]], axis=1) * r
    o_ref[...] = (
        jnp.dot(x_ref[...], w_ref[0:D_FEAT], preferred_element_type=jnp.float32)
        + jnp.dot(h, w_ref[D_FEAT:], preferred_element_type=jnp.float32)
        + b_ref[...]
    )


_BLK = 1000


def kernel(x, edge_index, W, b):
    src = edge_index[0]
    dst = edge_index[1]

    owner = dst // NB
    order = jnp.argsort(owner, stable=True)
    src_s = src[order]
    dst_s = dst[order]
    owner_s = owner[order]
    ends = jnp.searchsorted(owner_s, jnp.arange(1, NBK + 1), side="left")
    starts = jnp.concatenate([jnp.zeros((1,), ends.dtype), ends[:-1]])
    nch = -(-(ends - starts) // CHUNK)
    padded = nch * CHUNK
    base = jnp.concatenate([jnp.zeros((1,), padded.dtype),
                            jnp.cumsum(padded)[:-1]]).astype(jnp.int32)
    pos = base[owner_s] + (jnp.arange(N_EDGES, dtype=jnp.int32)
                           - starts[owner_s].astype(jnp.int32))
    slot_bucket = jnp.searchsorted(jnp.cumsum(padded), jnp.arange(E_CAP),
                                   side="right").astype(jnp.int32)
    slot_bucket = jnp.clip(slot_bucket, 0, NBK - 1)
    dst_local = dst_s - (owner_s // NS) * NPC
    src_p = jnp.zeros((E_CAP,), jnp.int32).at[pos].set(src_s * 2)
    dst_p = (TRASHL + 8 * (slot_bucket % NS)).at[pos].set(dst_local)

    src2 = jnp.concatenate([src_p, src_p + 1])

    x2 = x.reshape(2 * N_NODES, H)
    zeros = jnp.zeros((CHUNK, H), jnp.float32)
    ones = jnp.ones((CHUNK, H), jnp.float32)

    meta_idx = jnp.arange(NBK) * 8
    nch8 = jnp.zeros((8 * NBK + 16,), jnp.int32).at[meta_idx].set(
        nch.astype(jnp.int32))
    base8 = jnp.zeros((8 * NBK + 16,), jnp.int32).at[meta_idx].set(base)
    accs = _build_sc_scatter()(x2, src2, dst_p, nch8, base8, zeros, ones)

    grid = N_NODES // _BLK
    out = pl.pallas_call(
        _tc_body,
        grid=(grid,),
        in_specs=[
            pl.BlockSpec((_BLK, D_FEAT), lambda i: (i, 0)),
            pl.BlockSpec((3, _BLK, H), lambda i: (0, i, 0)),
            pl.BlockSpec((2 * D_FEAT, D_OUT), lambda i: (0, 0)),
            pl.BlockSpec((1, D_OUT), lambda i: (0, 0)),
        ],
        out_specs=pl.BlockSpec((_BLK, D_OUT), lambda i: (i, 0)),
        out_shape=jax.ShapeDtypeStruct((N_NODES, D_OUT), jnp.float32),
    )(x, accs, W, b.reshape(1, D_OUT))
    return out


# speedup vs baseline: 1.8571x; 1.8571x over previous
import functools

import jax
import jax.numpy as jnp
from jax import lax
from jax.experimental import pallas as pl
from jax.experimental.pallas import tpu as pltpu
from jax.experimental.pallas import tpu_sc as plsc

N_NODES = 10000
N_EDGES = 160000
D_FEAT = 256
D_OUT = 256
H = 128

NC = 2
NS = 16
NBK = NC * NS
CHUNK = 128
E_CAP = 164096
NB = 320
NP = 10240
NPC = NP // NC
TRASHL = NPC
NPCA = NPC + NS * 8


def _sc_body(x2_hbm, src2_hbm, dst_hbm, nch_hbm, base_hbm, zeros_hbm,
             ones_hbm, out_hbm,
             nchv, basev, srcv, dstv, rows, acc, sem):
    c = lax.axis_index("c")
    s = lax.axis_index("s")
    bucket = c * NS + s
    pltpu.sync_copy(nch_hbm, nchv)
    pltpu.sync_copy(base_hbm, basev)
    nch = nchv[pl.ds(bucket * 8, 16)][0]
    ebase = pl.multiple_of(basev[pl.ds(bucket * 8, 16)][0], CHUNK)

    for q in range(3):
        pltpu.sync_copy(zeros_hbm, rows)
        for z in range(NB // CHUNK):
            pltpu.sync_copy(rows, acc.at[pl.ds(s * NB + z * CHUNK, CHUNK)])
        pltpu.sync_copy(rows.at[pl.ds(0, NB % CHUNK)],
                        acc.at[pl.ds(s * NB + NB - NB % CHUNK, NB % CHUNK)])

        if q == 2:
            pltpu.sync_copy(ones_hbm, rows)

        def body(j, carry):
            off = ebase + j * CHUNK
            pltpu.sync_copy(dst_hbm.at[pl.ds(off, CHUNK)], dstv)
            if q < 2:
                pltpu.sync_copy(src2_hbm.at[pl.ds(q * E_CAP + off, CHUNK)],
                                srcv)
                pltpu.async_copy(x2_hbm.at[srcv], rows, sem).wait()
            pltpu.sync_copy(rows, acc.at[dstv], add=True)
            return carry

        lax.fori_loop(0, nch, body, 0)

        pltpu.sync_copy(acc.at[pl.ds(s * NB, NB)],
                        out_hbm.at[q, pl.ds(c * NPC + s * NB, NB)])


@functools.cache
def _build_sc_scatter():
    mesh = plsc.VectorSubcoreMesh(core_axis_name="c", subcore_axis_name="s",
                                  num_cores=NC, num_subcores=NS)
    return pl.kernel(
        _sc_body,
        out_type=jax.ShapeDtypeStruct((3, NP, H), jnp.float32),
        mesh=mesh,
        scratch_types=(
            pltpu.VMEM((8 * NBK + 16,), jnp.int32),
            pltpu.VMEM((8 * NBK + 16,), jnp.int32),
            pltpu.VMEM((CHUNK,), jnp.int32),
            pltpu.VMEM((CHUNK,), jnp.int32),
            pltpu.VMEM((CHUNK, H), jnp.float32),
            pltpu.VMEM_SHARED((NPCA, H), jnp.float32),
            pltpu.SemaphoreType.DMA,
        ),
    )


def _tc_body(x_ref, ns_ref, w_ref, b_ref, o_ref):
    r = 1.0 / jnp.maximum(ns_ref[2, :, 0:1], 1.0)
    h = jnp.concatenate([ns_ref[0], ns_ref[1]], axis=1) * r
    o_ref[...] = (
        jnp.dot(x_ref[...], w_ref[0:D_FEAT], preferred_element_type=jnp.float32)
        + jnp.dot(h, w_ref[D_FEAT:], preferred_element_type=jnp.float32)
        + b_ref[...]
    )


_BLK = 1000


def kernel(x, edge_index, W, b):
    src = edge_index[0]
    dst = edge_index[1]

    owner = dst // NB
    order = jnp.argsort(owner, stable=True)
    owner_s = owner[order]
    ends = jnp.searchsorted(owner_s, jnp.arange(1, NBK + 1), side="left")
    starts = jnp.concatenate([jnp.zeros((1,), ends.dtype), ends[:-1]])
    nch = -(-(ends - starts) // CHUNK)
    padded = nch * CHUNK
    base = jnp.concatenate([jnp.zeros((1,), padded.dtype),
                            jnp.cumsum(padded)[:-1]]).astype(jnp.int32)
    slots = jnp.arange(E_CAP, dtype=jnp.int32)
    slot_bucket = jnp.searchsorted(jnp.cumsum(padded), slots,
                                   side="right").astype(jnp.int32)
    slot_bucket = jnp.clip(slot_bucket, 0, NBK - 1)
    idx_e = starts[slot_bucket].astype(jnp.int32) + (slots - base[slot_bucket])
    valid = idx_e < ends[slot_bucket].astype(jnp.int32)
    oe = order[jnp.where(valid, idx_e, 0)]
    dst_local = dst[oe] - (owner[oe] // NS) * NPC
    src_p = jnp.where(valid, src[oe] * 2, 0).astype(jnp.int32)
    dst_p = jnp.where(valid, dst_local,
                      TRASHL + 8 * (slot_bucket % NS)).astype(jnp.int32)

    src2 = jnp.concatenate([src_p, src_p + 1])

    x2 = x.reshape(2 * N_NODES, H)
    zeros = jnp.zeros((CHUNK, H), jnp.float32)
    ones = jnp.ones((CHUNK, H), jnp.float32)

    meta_idx = jnp.arange(NBK) * 8
    nch8 = jnp.zeros((8 * NBK + 16,), jnp.int32).at[meta_idx].set(
        nch.astype(jnp.int32))
    base8 = jnp.zeros((8 * NBK + 16,), jnp.int32).at[meta_idx].set(base)
    accs = _build_sc_scatter()(x2, src2, dst_p, nch8, base8, zeros, ones)

    grid = N_NODES // _BLK
    out = pl.pallas_call(
        _tc_body,
        grid=(grid,),
        in_specs=[
            pl.BlockSpec((_BLK, D_FEAT), lambda i: (i, 0)),
            pl.BlockSpec((3, _BLK, H), lambda i: (0, i, 0)),
            pl.BlockSpec((2 * D_FEAT, D_OUT), lambda i: (0, 0)),
            pl.BlockSpec((1, D_OUT), lambda i: (0, 0)),
        ],
        out_specs=pl.BlockSpec((_BLK, D_OUT), lambda i: (i, 0)),
        out_shape=jax.ShapeDtypeStruct((N_NODES, D_OUT), jnp.float32),
    )(x, accs, W, b.reshape(1, D_OUT))
    return out
